# Initial kernel scaffold; baseline (speedup 1.0000x reference)
import jax
import jax.numpy as jnp
from jax.experimental import pallas as pl


def kernel(node_species, embed_table, W):
    raise NotImplementedError("write your pallas kernel here")


# speedup vs baseline: 1.6250x; 1.6250x over previous
import functools

import jax
import jax.numpy as jnp
from jax import lax
from jax.experimental import pallas as pl
from jax.experimental.pallas import tpu as pltpu
from jax.experimental.pallas import tpu_sc as plsc

NUM_NODES = 100000
EMBED_DIM = 128
OUT_DIM = 128

_NC = 2
_NS = 16
_NW = _NC * _NS

_CHUNK = 160
_NUM_CHUNKS = NUM_NODES // _CHUNK
_STEPS = -(-_NUM_CHUNKS // _NW)


def _project_body(e_ref, w_ref, p_ref):
    p_ref[...] = jnp.dot(e_ref[...], w_ref[...],
                         preferred_element_type=jnp.float32)


def _project(embed_table, W):
    return pl.pallas_call(
        _project_body,
        out_shape=jax.ShapeDtypeStruct(
            (embed_table.shape[0], W.shape[1]), jnp.float32),
    )(embed_table, W)


def _gather_body(idx_hbm, p_hbm, out_hbm, idx_v, rows_v, sem):
    wid = lax.axis_index("s") * _NC + lax.axis_index("c")

    def step(k, carry):
        cid = wid + k * _NW

        @pl.when(cid < _NUM_CHUNKS)
        def _():
            base = cid * _CHUNK
            pltpu.sync_copy(idx_hbm.at[pl.ds(base, _CHUNK)], idx_v)
            pltpu.async_copy(p_hbm.at[idx_v], rows_v, sem).wait()
            pltpu.sync_copy(rows_v, out_hbm.at[pl.ds(base, _CHUNK)])

        return carry

    lax.fori_loop(0, _STEPS, step, 0)


@functools.partial(jax.jit, static_argnames=())
def _gather(node_species, p):
    mesh = plsc.VectorSubcoreMesh(core_axis_name="c", subcore_axis_name="s")
    return pl.kernel(
        _gather_body,
        out_type=jax.ShapeDtypeStruct((NUM_NODES, OUT_DIM), jnp.float32),
        mesh=mesh,
        scratch_types=[
            pltpu.VMEM((_CHUNK,), jnp.int32),
            pltpu.VMEM((_CHUNK, OUT_DIM), jnp.float32),
            pltpu.SemaphoreType.DMA,
        ],
    )(node_species, p)


def kernel(node_species, embed_table, W):
    p = _project(embed_table, W)
    return _gather(node_species.astype(jnp.int32), p)


# speedup vs baseline: 1.6752x; 1.0309x over previous
import jax
import jax.numpy as jnp
from jax import lax
from jax.experimental import pallas as pl
from jax.experimental.pallas import tpu as pltpu
from jax.experimental.pallas import tpu_sc as plsc

NUM_NODES = 100000
EMBED_DIM = 128
OUT_DIM = 128

_NC = 2
_NS = 16
_NW = _NC * _NS

_CHUNK = 160
_NUM_CHUNKS = NUM_NODES // _CHUNK
_BASE_N = _NUM_CHUNKS // _NW
_EXTRA = _NUM_CHUNKS - _BASE_N * _NW
_MAX_N = _BASE_N + 1
_NBUF = 4
_NG = -(-_MAX_N // _NBUF)


def _project_body(e_ref, w_ref, p_ref):
    p_ref[...] = jnp.dot(e_ref[...], w_ref[...],
                         preferred_element_type=jnp.float32)


def _project(embed_table, W):
    return pl.pallas_call(
        _project_body,
        out_shape=jax.ShapeDtypeStruct(
            (embed_table.shape[0], W.shape[1]), jnp.float32),
    )(embed_table, W)


def _gather_body(idx_hbm, p_hbm, out_hbm, idx_v, bufs, gsems, wsems):
    wid = lax.axis_index("s") * _NC + lax.axis_index("c")
    n = _BASE_N + (wid < _EXTRA).astype(jnp.int32)
    start = wid * _BASE_N + jnp.minimum(wid, _EXTRA)
    row0 = start * _CHUNK

    pltpu.sync_copy(idx_hbm.at[pl.ds(row0, _BASE_N * _CHUNK)],
                    idx_v.at[pl.ds(0, _BASE_N * _CHUNK)])

    @pl.when(n > _BASE_N)
    def _():
        pltpu.sync_copy(
            idx_hbm.at[pl.ds(row0 + _BASE_N * _CHUNK, _CHUNK)],
            idx_v.at[pl.ds(_BASE_N * _CHUNK, _CHUNK)])

    def gstart(l, b):
        pltpu.async_copy(
            p_hbm.at[idx_v.at[pl.ds(l * _CHUNK, _CHUNK)]], bufs[b], gsems[b])

    def gwait(b):
        pltpu.make_async_copy(
            p_hbm.at[idx_v.at[pl.ds(0, _CHUNK)]], bufs[b], gsems[b]).wait()

    def wstart(l, b):
        pltpu.async_copy(
            bufs[b], out_hbm.at[pl.ds(row0 + l * _CHUNK, _CHUNK)], wsems[b])

    def wwait(b):
        pltpu.make_async_copy(
            bufs[b], out_hbm.at[pl.ds(row0, _CHUNK)], wsems[b]).wait()

    for b in range(_NBUF):
        gstart(b, b)

    for g in range(_NG):
        for b in range(_NBUF):
            l = g * _NBUF + b
            if l < _MAX_N:
                @pl.when(l < n)
                def _(l=l, b=b):
                    gwait(b)
                    wstart(l, b)
        for b in range(_NBUF):
            ln = (g + 1) * _NBUF + b
            if ln < _MAX_N:
                @pl.when(ln < n)
                def _(ln=ln, b=b):
                    wwait(b)
                    gstart(ln, b)

    for b in range(_NBUF):
        wwait(b)


def _gather(node_species, p):
    mesh = plsc.VectorSubcoreMesh(core_axis_name="c", subcore_axis_name="s")
    return pl.kernel(
        _gather_body,
        out_type=jax.ShapeDtypeStruct((NUM_NODES, OUT_DIM), jnp.float32),
        mesh=mesh,
        scratch_types=[
            pltpu.VMEM((_MAX_N * _CHUNK,), jnp.int32),
            [pltpu.VMEM((_CHUNK, OUT_DIM), jnp.float32)
             for _ in range(_NBUF)],
            [pltpu.SemaphoreType.DMA for _ in range(_NBUF)],
            [pltpu.SemaphoreType.DMA for _ in range(_NBUF)],
        ],
    )(node_species, p)


def kernel(node_species, embed_table, W):
    p = _project(embed_table, W)
    return _gather(node_species.astype(jnp.int32), p)


# speedup vs baseline: 5.7915x; 3.4571x over previous
import jax
import jax.numpy as jnp
from jax import lax
from jax.experimental import pallas as pl
from jax.experimental.pallas import tpu as pltpu
from jax.experimental.pallas import tpu_sc as plsc

NUM_NODES = 100000
EMBED_DIM = 128
OUT_DIM = 128

_NC = 2
_NS = 16
_NW = _NC * _NS
_LANES = 16

_CHUNK = 160
_NGROUP = _CHUNK // _LANES
_NUM_CHUNKS = NUM_NODES // _CHUNK
_BASE_N = _NUM_CHUNKS // _NW
_EXTRA = _NUM_CHUNKS - _BASE_N * _NW
_MAX_N = _BASE_N + 1
_NBUF = 4


def _project_body(e_ref, w_ref, p_ref):
    p_ref[...] = jnp.dot(e_ref[...], w_ref[...],
                         preferred_element_type=jnp.float32)


def _project(embed_table, W):
    return pl.pallas_call(
        _project_body,
        out_shape=jax.ShapeDtypeStruct(
            (embed_table.shape[0], W.shape[1]), jnp.float32),
    )(embed_table, W)


def _gather_body(idx_hbm, p_hbm, out_hbm, p_v, idx_v, bufs, wsems):
    wid = lax.axis_index("s") * _NC + lax.axis_index("c")
    n = _BASE_N + (wid < _EXTRA).astype(jnp.int32)
    start = wid * _BASE_N + jnp.minimum(wid, _EXTRA)
    row0 = start * _CHUNK

    pltpu.sync_copy(p_hbm, p_v)
    pltpu.sync_copy(idx_hbm.at[pl.ds(row0, _BASE_N * _CHUNK)],
                    idx_v.at[pl.ds(0, _BASE_N * _CHUNK)])

    @pl.when(n > _BASE_N)
    def _():
        pltpu.sync_copy(
            idx_hbm.at[pl.ds(row0 + _BASE_N * _CHUNK, _CHUNK)],
            idx_v.at[pl.ds(_BASE_N * _CHUNK, _CHUNK)])

    lane = lax.iota(jnp.int32, _LANES)
    cols = [j * _LANES + lane for j in range(OUT_DIM // _LANES)]

    def fill(l, b):
        buf = bufs[b]

        def group(g, carry):
            rows = idx_v[pl.ds(l * _CHUNK + g * _LANES, _LANES)]
            rbase = rows * OUT_DIM
            obase = (g * _LANES + lane) * OUT_DIM
            for c in cols:
                vals = plsc.load_gather(p_v, [rbase + c])
                plsc.store_scatter(buf, [obase + c], vals)
            return carry

        lax.fori_loop(0, _NGROUP, group, 0)

    def wstart(l, b):
        pltpu.async_copy(
            bufs[b],
            out_hbm.at[pl.ds((row0 + l * _CHUNK) * OUT_DIM, _CHUNK * OUT_DIM)],
            wsems[b])

    def wwait(b):
        pltpu.make_async_copy(
            bufs[b], out_hbm.at[pl.ds(0, _CHUNK * OUT_DIM)], wsems[b]).wait()

    for l in range(_MAX_N):
        b = l % _NBUF

        @pl.when(l < n)
        def _(l=l, b=b):
            if l >= _NBUF:
                wwait(b)
            fill(l, b)
            wstart(l, b)

    for b in range(_NBUF):
        wwait(b)


def _gather(node_species, p_flat, num_species):
    mesh = plsc.VectorSubcoreMesh(core_axis_name="c", subcore_axis_name="s")
    return pl.kernel(
        _gather_body,
        out_type=jax.ShapeDtypeStruct((NUM_NODES * OUT_DIM,), jnp.float32),
        mesh=mesh,
        compiler_params=pltpu.CompilerParams(needs_layout_passes=False),
        scratch_types=[
            pltpu.VMEM((num_species * OUT_DIM,), jnp.float32),
            pltpu.VMEM((_MAX_N * _CHUNK,), jnp.int32),
            [pltpu.VMEM((_CHUNK * OUT_DIM,), jnp.float32)
             for _ in range(_NBUF)],
            [pltpu.SemaphoreType.DMA for _ in range(_NBUF)],
        ],
    )(node_species, p_flat)


def kernel(node_species, embed_table, W):
    p = _project(embed_table, W)
    out_flat = _gather(node_species.astype(jnp.int32),
                       p.reshape(-1), embed_table.shape[0])
    return out_flat.reshape(NUM_NODES, OUT_DIM)
